# Initial kernel scaffold; baseline (speedup 1.0000x reference)
import functools

import jax
import jax.numpy as jnp
from jax import lax
from jax.experimental import pallas as pl
from jax.experimental.pallas import tpu as pltpu
from jax.experimental.pallas import tpu_sc as plsc

DIM = 64
ROWS_IDX = 128
K = 8
CH = ROWS_IDX * K
NC = 2
NS = 16
NW = NC * NS


@functools.lru_cache(maxsize=None)
def _make_kernel(B):
    b_per_w = B // NW
    n_iter = b_per_w // CH
    mesh = plsc.VectorSubcoreMesh(core_axis_name="c", subcore_axis_name="s")

    @functools.partial(
        pl.kernel,
        out_type=jax.ShapeDtypeStruct((B, DIM), jnp.float32),
        mesh=mesh,
        scratch_types=[
            pltpu.VMEM((K, ROWS_IDX), jnp.int32),
            pltpu.VMEM((CH, DIM), jnp.float32),
            pltpu.SemaphoreType.DMA,
        ],
    )
    def emb_kernel(idx_hbm, table_hbm, out_hbm, idx_v, rows_v, sem):
        wid = lax.axis_index("s") * NC + lax.axis_index("c")
        row_base = wid * (b_per_w // ROWS_IDX)
        base = wid * b_per_w

        def body(i, carry):
            pltpu.sync_copy(idx_hbm.at[pl.ds(row_base + i * K, K)], idx_v)
            copies = [
                pltpu.async_copy(
                    table_hbm.at[idx_v.at[j]],
                    rows_v.at[pl.ds(j * ROWS_IDX, ROWS_IDX)],
                    sem,
                )
                for j in range(K)
            ]
            for c in copies:
                c.wait()
            pltpu.sync_copy(rows_v, out_hbm.at[pl.ds(base + i * CH, CH)])
            return carry

        lax.fori_loop(0, n_iter, body, 0)

    return emb_kernel


@jax.jit
def kernel(input, table):
    S, T = input.shape
    B = S * T
    idx2d = input.reshape(B // ROWS_IDX, ROWS_IDX).astype(jnp.int32)
    out = _make_kernel(B)(idx2d, table)
    return out.reshape(S, T, DIM)


# speedup vs baseline: 4.9690x; 4.9690x over previous
import functools

import jax
import jax.numpy as jnp
from jax import lax
from jax.experimental import pallas as pl
from jax.experimental.pallas import tpu as pltpu
from jax.experimental.pallas import tpu_sc as plsc

DIM = 64
ROWS_IDX = 128
K = 8
CH = ROWS_IDX * K
NC = 2
NS = 16
NW = NC * NS


@functools.lru_cache(maxsize=None)
def _make_kernel(B):
    b_per_w = B // NW
    n_iter = b_per_w // CH
    mesh = plsc.VectorSubcoreMesh(core_axis_name="c", subcore_axis_name="s")

    @functools.partial(
        pl.kernel,
        out_type=jax.ShapeDtypeStruct((B, DIM), jnp.float32),
        mesh=mesh,
        scratch_types=[
            pltpu.VMEM((K, ROWS_IDX), jnp.int32),
            pltpu.VMEM((CH, DIM), jnp.float32),
            pltpu.SemaphoreType.DMA,
        ],
        compiler_params=pltpu.CompilerParams(use_tc_tiling_on_sc=False),
    )
    def emb_kernel(idx_hbm, table_hbm, out_hbm, idx_v, rows_v, sem):
        wid = lax.axis_index("s") * NC + lax.axis_index("c")
        row_base = wid * (b_per_w // ROWS_IDX)
        base = wid * b_per_w

        def body(i, carry):
            pltpu.sync_copy(idx_hbm.at[pl.ds(row_base + i * K, K)], idx_v)
            copies = [
                pltpu.async_copy(
                    table_hbm.at[idx_v.at[j]],
                    rows_v.at[pl.ds(j * ROWS_IDX, ROWS_IDX)],
                    sem,
                )
                for j in range(K)
            ]
            for c in copies:
                c.wait()
            pltpu.sync_copy(rows_v, out_hbm.at[pl.ds(base + i * CH, CH)])
            return carry

        lax.fori_loop(0, n_iter, body, 0)

    return emb_kernel


@jax.jit
def kernel(input, table):
    S, T = input.shape
    B = S * T
    idx2d = input.reshape(B // ROWS_IDX, ROWS_IDX).astype(jnp.int32)
    out = _make_kernel(B)(idx2d, table)
    return out.reshape(S, T, DIM)


# speedup vs baseline: 5.1341x; 1.0332x over previous
import functools

import jax
import jax.numpy as jnp
from jax import lax
from jax.experimental import pallas as pl
from jax.experimental.pallas import tpu as pltpu
from jax.experimental.pallas import tpu_sc as plsc

DIM = 64
ROWS_IDX = 128
K = 4
CH = ROWS_IDX * K
NC = 2
NS = 16
NW = NC * NS


@functools.lru_cache(maxsize=None)
def _make_kernel(B):
    b_per_w = B // NW
    n_iter = b_per_w // CH
    n2 = n_iter // 2
    mesh = plsc.VectorSubcoreMesh(core_axis_name="c", subcore_axis_name="s")

    @functools.partial(
        pl.kernel,
        out_type=jax.ShapeDtypeStruct((B, DIM), jnp.float32),
        mesh=mesh,
        scratch_types=[
            pltpu.VMEM((K, ROWS_IDX), jnp.int32),
            pltpu.VMEM((K, ROWS_IDX), jnp.int32),
            pltpu.VMEM((CH, DIM), jnp.float32),
            pltpu.VMEM((CH, DIM), jnp.float32),
            pltpu.SemaphoreType.DMA,
            pltpu.SemaphoreType.DMA,
            pltpu.SemaphoreType.DMA,
            pltpu.SemaphoreType.DMA,
        ],
        compiler_params=pltpu.CompilerParams(use_tc_tiling_on_sc=False),
    )
    def emb_kernel(idx_hbm, table_hbm, out_hbm, i0, i1, r0, r1, gs0, gs1,
                   ss0, ss1):
        wid = lax.axis_index("s") * NC + lax.axis_index("c")
        row_base = wid * (b_per_w // ROWS_IDX)
        base = wid * b_per_w

        def load_idx(c, iv):
            pltpu.sync_copy(idx_hbm.at[pl.ds(row_base + c * K, K)], iv)

        def fire_gathers(iv, rv, sem):
            for j in range(K):
                pltpu.async_copy(
                    table_hbm.at[iv.at[j]],
                    rv.at[pl.ds(j * ROWS_IDX, ROWS_IDX)],
                    sem,
                )

        def wait_gathers(iv, rv, sem):
            for j in range(K):
                pltpu.make_async_copy(
                    table_hbm.at[iv.at[j]],
                    rv.at[pl.ds(j * ROWS_IDX, ROWS_IDX)],
                    sem,
                ).wait()

        def start_store(rv, c, sem):
            pltpu.async_copy(rv, out_hbm.at[pl.ds(base + c * CH, CH)], sem)

        def wait_store(rv, c, sem):
            pltpu.make_async_copy(
                rv, out_hbm.at[pl.ds(base + c * CH, CH)], sem
            ).wait()

        load_idx(0, i0)
        fire_gathers(i0, r0, gs0)

        def body(s, carry):
            c0 = 2 * s
            c1 = c0 + 1
            load_idx(c1, i1)

            @pl.when(s >= 1)
            def _():
                wait_store(r1, c1 - 2, ss1)

            fire_gathers(i1, r1, gs1)
            wait_gathers(i0, r0, gs0)
            start_store(r0, c0, ss0)

            @pl.when(s < n2 - 1)
            def _():
                load_idx(c0 + 2, i0)
                wait_store(r0, c0, ss0)
                fire_gathers(i0, r0, gs0)

            wait_gathers(i1, r1, gs1)
            start_store(r1, c1, ss1)
            return carry

        lax.fori_loop(0, n2, body, 0)
        wait_store(r0, n_iter - 2, ss0)
        wait_store(r1, n_iter - 1, ss1)

    return emb_kernel


@jax.jit
def kernel(input, table):
    S, T = input.shape
    B = S * T
    idx2d = input.reshape(B // ROWS_IDX, ROWS_IDX).astype(jnp.int32)
    out = _make_kernel(B)(idx2d, table)
    return out.reshape(S, T, DIM)


# speedup vs baseline: 5.7931x; 1.1284x over previous
import functools

import jax
import jax.numpy as jnp
from jax import lax
from jax.experimental import pallas as pl
from jax.experimental.pallas import tpu as pltpu
from jax.experimental.pallas import tpu_sc as plsc

DIM = 64
ROWS_IDX = 128
K = 4
CH = ROWS_IDX * K
NC = 2
NS = 16
NW = NC * NS


V = 10000
V_PER_TILE = V // NS


@functools.lru_cache(maxsize=None)
def _make_kernel(B):
    b_per_w = B // NW
    n_iter = b_per_w // CH
    n2 = n_iter // 2
    mesh = plsc.VectorSubcoreMesh(core_axis_name="c", subcore_axis_name="s")

    @functools.partial(
        pl.kernel,
        out_type=jax.ShapeDtypeStruct((B, DIM), jnp.float32),
        mesh=mesh,
        scratch_types=[
            pltpu.VMEM_SHARED((V, DIM), jnp.float32),
            pltpu.VMEM((K, ROWS_IDX), jnp.int32),
            pltpu.VMEM((K, ROWS_IDX), jnp.int32),
            pltpu.VMEM((CH, DIM), jnp.float32),
            pltpu.VMEM((CH, DIM), jnp.float32),
            pltpu.SemaphoreType.DMA,
            pltpu.SemaphoreType.DMA,
            pltpu.SemaphoreType.DMA,
            pltpu.SemaphoreType.DMA,
        ],
        compiler_params=pltpu.CompilerParams(use_tc_tiling_on_sc=False),
    )
    def emb_kernel(idx_hbm, table_hbm, out_hbm, tab_sp, i0, i1, r0, r1,
                   gs0, gs1, ss0, ss1):
        sid = lax.axis_index("s")
        wid = sid * NC + lax.axis_index("c")
        row_base = wid * (b_per_w // ROWS_IDX)
        base = wid * b_per_w

        pltpu.sync_copy(
            table_hbm.at[pl.ds(sid * V_PER_TILE, V_PER_TILE)],
            tab_sp.at[pl.ds(sid * V_PER_TILE, V_PER_TILE)],
        )
        plsc.subcore_barrier()

        def load_idx(c, iv):
            pltpu.sync_copy(idx_hbm.at[pl.ds(row_base + c * K, K)], iv)

        def fire_gathers(iv, rv, sem):
            for j in range(K):
                pltpu.async_copy(
                    tab_sp.at[iv.at[j]],
                    rv.at[pl.ds(j * ROWS_IDX, ROWS_IDX)],
                    sem,
                )

        def wait_gathers(iv, rv, sem):
            for j in range(K):
                pltpu.make_async_copy(
                    tab_sp.at[iv.at[j]],
                    rv.at[pl.ds(j * ROWS_IDX, ROWS_IDX)],
                    sem,
                ).wait()

        def start_store(rv, c, sem):
            pltpu.async_copy(rv, out_hbm.at[pl.ds(base + c * CH, CH)], sem)

        def wait_store(rv, c, sem):
            pltpu.make_async_copy(
                rv, out_hbm.at[pl.ds(base + c * CH, CH)], sem
            ).wait()

        load_idx(0, i0)
        fire_gathers(i0, r0, gs0)

        def body(s, carry):
            c0 = 2 * s
            c1 = c0 + 1
            load_idx(c1, i1)

            @pl.when(s >= 1)
            def _():
                wait_store(r1, c1 - 2, ss1)

            fire_gathers(i1, r1, gs1)
            wait_gathers(i0, r0, gs0)
            start_store(r0, c0, ss0)

            @pl.when(s < n2 - 1)
            def _():
                load_idx(c0 + 2, i0)
                wait_store(r0, c0, ss0)
                fire_gathers(i0, r0, gs0)

            wait_gathers(i1, r1, gs1)
            start_store(r1, c1, ss1)
            return carry

        lax.fori_loop(0, n2, body, 0)
        wait_store(r0, n_iter - 2, ss0)
        wait_store(r1, n_iter - 1, ss1)

    return emb_kernel


@jax.jit
def kernel(input, table):
    S, T = input.shape
    B = S * T
    idx2d = input.reshape(B // ROWS_IDX, ROWS_IDX).astype(jnp.int32)
    out = _make_kernel(B)(idx2d, table)
    return out.reshape(S, T, DIM)
